# baseline (speedup 1.0000x reference)
import functools

import jax
import jax.numpy as jnp
from jax import lax
from jax.experimental import pallas as pl
from jax.experimental.pallas import tpu as pltpu
from jax.experimental.pallas import tpu_sc as plsc

_B, _NPOS, _NNEG, _D, _NLEM = 8, 128, 64, 128, 16384
_MOM = 0.5
_T = 2048
_GRID = _NLEM // _T

_NW = 32
_HPW = 16
_CPW = (_B * _NNEG) // _NW
_PLAIN0 = _NPOS + _B * _NNEG
_PPW = 496
_L = 16


def _tc_body(x3_ref, xneg_ref, vis_ref, mem_ref, sim_ref, noise_ref, acc_ref):
    i = pl.program_id(0)
    x3 = x3_ref[...]
    xf = x3.reshape(_B * _NPOS, _D)
    mem = mem_ref[...]
    sim_ref[...] = jax.lax.dot_general(
        xf, mem, (((1,), (1,)), ((), ())), preferred_element_type=jnp.float32)

    @pl.when(i == 0)
    def _():
        vis = vis_ref[...]
        noise_ref[...] = jax.lax.dot_general(
            xneg_ref[...], mem[0:_NPOS, :], (((1,), (1,)), ((), ())),
            preferred_element_type=jnp.float32)
        acc_ref[...] = jnp.sum((vis > 0).astype(jnp.int32), axis=0,
                               keepdims=True)


def _tc_call(x3, xneg, visible, memory):
    return pl.pallas_call(
        _tc_body,
        grid=(_GRID,),
        in_specs=[
            pl.BlockSpec((_B, _NPOS, _D), lambda i: (0, 0, 0)),
            pl.BlockSpec((_B * _NNEG, _D), lambda i: (0, 0)),
            pl.BlockSpec((_B, _NPOS), lambda i: (0, 0)),
            pl.BlockSpec((_T, _D), lambda i: (i, 0)),
        ],
        out_specs=[
            pl.BlockSpec((_B * _NPOS, _T), lambda i: (0, i)),
            pl.BlockSpec((_B * _NNEG, _NPOS), lambda i: (0, 0)),
            pl.BlockSpec((1, _NPOS), lambda i: (0, 0)),
        ],
        out_shape=[
            jax.ShapeDtypeStruct((_B * _NPOS, _NLEM), jnp.float32),
            jax.ShapeDtypeStruct((_B * _NNEG, _NPOS), jnp.float32),
            jax.ShapeDtypeStruct((1, _NPOS), jnp.int32),
        ],
    )(x3, xneg, visible, memory)


def _row_scale(s_v, fold_buf, j):
    t = jnp.broadcast_to(s_v[0], (_L,))
    for k in range(1, _L):
        t = t + jnp.broadcast_to(s_v[k], (_L,))
    fold_buf[16 * j:16 * j + 16] = t
    s = fold_buf[16 * j:16 * j + 16][0]
    i = lax.bitcast_convert_type(s, jnp.int32)
    i = jnp.int32(0x5F3759DF) - (i >> 1)
    y = lax.bitcast_convert_type(i, jnp.float32)
    for _ in range(3):
        y = y * (1.5 - 0.5 * s * y * y)
    return jnp.minimum(y, 1e12)


_BLK = 8


def _renorm_rows(buf, n, fold_buf):
    def body(i, _):
        r0 = i * _BLK
        svs = []
        for j in range(_BLK):
            s_v = None
            for k in range(8):
                v = buf[r0 + j, 16 * k:16 * k + 16]
                s_v = v * v if s_v is None else s_v + v * v
            svs.append(s_v)
        invs = [_row_scale(svs[j], fold_buf, j) for j in range(_BLK)]
        for j in range(_BLK):
            for k in range(8):
                buf[r0 + j, 16 * k:16 * k + 16] = (
                    buf[r0 + j, 16 * k:16 * k + 16] * invs[j])
        return 0
    lax.fori_loop(0, n // _BLK, body, 0)


def _sc_body(x_hbm, vis_hbm, mem_hbm, out_hbm,
             plain_buf, clut_buf, memh_buf, xh_buf, vis_buf, fold_buf, sem):
    wid = lax.axis_index("s") * 2 + lax.axis_index("c")
    h0 = (wid % 8) * _HPW
    c0 = _NPOS + wid * _CPW
    p0 = _PLAIN0 + 488 * wid + 8 * jnp.maximum(0, wid - 16)
    cb = wid // 4
    cp = _NPOS + (wid % 4) * _CPW

    big = pltpu.async_copy(mem_hbm.at[pl.ds(p0, _PPW), :], plain_buf, sem)
    pltpu.sync_copy(mem_hbm.at[pl.ds(h0, _HPW), :], memh_buf)
    pltpu.sync_copy(x_hbm.at[:, pl.ds(h0, _HPW), :], xh_buf)
    pltpu.sync_copy(x_hbm.at[cb, pl.ds(cp, _CPW), :], clut_buf)
    pltpu.sync_copy(vis_hbm, vis_buf)

    for p in range(_HPW):
        for k in range(8):
            memh_buf[p, 16 * k:16 * k + 16] = (
                memh_buf[p, 16 * k:16 * k + 16] * _MOM)

    def bodyb(b, _):
        visv_b = vis_buf[b, pl.ds(h0, _L)]
        for p in range(_HPW):
            vsb = visv_b[p] * ((1.0 - _MOM) / _B)
            for k in range(8):
                memh_buf[p, 16 * k:16 * k + 16] = (
                    memh_buf[p, 16 * k:16 * k + 16]
                    + xh_buf[b, p, 16 * k:16 * k + 16] * vsb)
        return 0
    lax.fori_loop(0, _B, bodyb, 0)
    _renorm_rows(memh_buf, _HPW, fold_buf)
    _renorm_rows(clut_buf, _CPW, fold_buf)

    big.wait()
    _renorm_rows(plain_buf, _PPW, fold_buf)

    pltpu.sync_copy(memh_buf, out_hbm.at[pl.ds(h0, _HPW), :])
    pltpu.sync_copy(clut_buf, out_hbm.at[pl.ds(c0, _CPW), :])
    pltpu.sync_copy(plain_buf, out_hbm.at[pl.ds(p0, _PPW), :])


_sc_call = functools.partial(
    pl.kernel,
    out_type=jax.ShapeDtypeStruct((_NLEM, _D), jnp.float32),
    mesh=plsc.VectorSubcoreMesh(core_axis_name="c", subcore_axis_name="s"),
    scratch_types=[
        pltpu.VMEM((_PPW, _D), jnp.float32),
        pltpu.VMEM((_CPW, _D), jnp.float32),
        pltpu.VMEM((_HPW, _D), jnp.float32),
        pltpu.VMEM((_B, _HPW, _D), jnp.float32),
        pltpu.VMEM((_B, _NPOS), jnp.float32),
        pltpu.VMEM((_BLK * _L,), jnp.float32),
        pltpu.SemaphoreType.DMA,
    ],
)(_sc_body)


def kernel(x, y, visible, memory):
    x3 = x[:, 0:_NPOS, :]
    xneg = x[:, _NPOS:, :].reshape(_B * _NNEG, _D)

    sim, noise, acc = _tc_call(x3, xneg, visible, memory)
    new_memory = _sc_call(x, visible, memory)

    similarity = sim.reshape(_B, _NPOS, _NLEM)
    noise_similarity = noise.reshape(_B, _NNEG, _NPOS)
    y_idx = y.astype(jnp.int32)
    accumulate_delta = acc.reshape(_NPOS)
    return (similarity, y_idx, noise_similarity, new_memory, accumulate_delta)


# speedup vs baseline: 2.0366x; 2.0366x over previous
import jax
import jax.numpy as jnp
from jax.experimental import pallas as pl

_B, _NPOS, _NNEG, _D, _NLEM = 8, 128, 64, 128, 16384
_MOM = 0.5
_T = 2048
_GRID = _NLEM // _T


def _renorm(m):
    s = jnp.sum(m * m, axis=1, keepdims=True)
    return m / jnp.maximum(jnp.sqrt(s), 1e-12)


def _body(x_ref, vis_ref, mem_ref, sim_ref, noise_ref, newmem_ref, acc_ref):
    i = pl.program_id(0)
    x3 = x_ref[:, 0:_NPOS, :]
    xf = x3.reshape(_B * _NPOS, _D)
    mem = mem_ref[...]
    sim_ref[...] = jax.lax.dot_general(
        xf, mem, (((1,), (1,)), ((), ())), preferred_element_type=jnp.float32)

    @pl.when(i == 0)
    def _():
        vis = vis_ref[...]
        xneg = x_ref[:, _NPOS:, :].reshape(_B * _NNEG, _D)
        mem_head = mem[0:_NPOS, :]
        noise_ref[...] = jax.lax.dot_general(
            xneg, mem_head, (((1,), (1,)), ((), ())),
            preferred_element_type=jnp.float32)
        get = jnp.mean(x3 * vis[..., None], axis=0)
        head = mem_head * _MOM + get * (1.0 - _MOM)
        newmem_ref[0:_NPOS, :] = _renorm(head)
        newmem_ref[_NPOS:_NPOS + _B * _NNEG, :] = _renorm(xneg)
        newmem_ref[_NPOS + _B * _NNEG:, :] = _renorm(mem[_NPOS + _B * _NNEG:, :])
        acc_ref[...] = jnp.sum((vis > 0).astype(jnp.int32), axis=0,
                               keepdims=True)

    @pl.when(i != 0)
    def _():
        newmem_ref[...] = _renorm(mem)


def kernel(x, y, visible, memory):
    sim, noise, new_memory, acc = pl.pallas_call(
        _body,
        grid=(_GRID,),
        in_specs=[
            pl.BlockSpec((_B, _NPOS + _NNEG, _D), lambda i: (0, 0, 0)),
            pl.BlockSpec((_B, _NPOS), lambda i: (0, 0)),
            pl.BlockSpec((_T, _D), lambda i: (i, 0)),
        ],
        out_specs=[
            pl.BlockSpec((_B * _NPOS, _T), lambda i: (0, i)),
            pl.BlockSpec((_B * _NNEG, _NPOS), lambda i: (0, 0)),
            pl.BlockSpec((_T, _D), lambda i: (i, 0)),
            pl.BlockSpec((1, _NPOS), lambda i: (0, 0)),
        ],
        out_shape=[
            jax.ShapeDtypeStruct((_B * _NPOS, _NLEM), jnp.float32),
            jax.ShapeDtypeStruct((_B * _NNEG, _NPOS), jnp.float32),
            jax.ShapeDtypeStruct((_NLEM, _D), jnp.float32),
            jax.ShapeDtypeStruct((1, _NPOS), jnp.int32),
        ],
    )(x, visible, memory)

    similarity = sim.reshape(_B, _NPOS, _NLEM)
    noise_similarity = noise.reshape(_B, _NNEG, _NPOS)
    y_idx = y.astype(jnp.int32)
    accumulate_delta = acc.reshape(_NPOS)
    return (similarity, y_idx, noise_similarity, new_memory, accumulate_delta)


# speedup vs baseline: 2.0646x; 1.0137x over previous
import jax
import jax.numpy as jnp
from jax.experimental import pallas as pl

_B, _NPOS, _NNEG, _D, _NLEM = 8, 128, 64, 128, 16384
_MOM = 0.5
_T = 2048
_GRID = _NLEM // _T


def _renorm(m):
    s = jnp.sum(m * m, axis=1, keepdims=True)
    return m / jnp.maximum(jnp.sqrt(s), 1e-12)


def _body(x_ref, y_ref, vis_ref, mem_ref,
          sim_ref, noise_ref, newmem_ref, acc_ref, y_out_ref):
    i = pl.program_id(0)
    x3 = x_ref[:, 0:_NPOS, :]
    xf = x3.reshape(_B * _NPOS, _D)
    mem = mem_ref[...]
    sim_ref[...] = jax.lax.dot_general(
        xf, mem, (((1,), (1,)), ((), ())), preferred_element_type=jnp.float32)

    @pl.when(i == 0)
    def _():
        vis = vis_ref[...]
        xneg = x_ref[:, _NPOS:, :].reshape(_B * _NNEG, _D)
        mem_head = mem[0:_NPOS, :]
        noise_ref[...] = jax.lax.dot_general(
            xneg, mem_head, (((1,), (1,)), ((), ())),
            preferred_element_type=jnp.float32)
        get = jnp.mean(x3 * vis[..., None], axis=0)
        head = mem_head * _MOM + get * (1.0 - _MOM)
        newmem_ref[0:_NPOS, :] = _renorm(head)
        newmem_ref[_NPOS:_NPOS + _B * _NNEG, :] = _renorm(xneg)
        newmem_ref[_NPOS + _B * _NNEG:, :] = _renorm(mem[_NPOS + _B * _NNEG:, :])
        acc_ref[...] = jnp.sum((vis > 0).astype(jnp.int32), axis=0,
                               keepdims=True)
        y_out_ref[...] = y_ref[...]

    @pl.when(i != 0)
    def _():
        newmem_ref[...] = _renorm(mem)


def kernel(x, y, visible, memory):
    sim, noise, new_memory, acc, y_idx = pl.pallas_call(
        _body,
        grid=(_GRID,),
        in_specs=[
            pl.BlockSpec((_B, _NPOS + _NNEG, _D), lambda i: (0, 0, 0)),
            pl.BlockSpec((_B, _NPOS), lambda i: (0, 0)),
            pl.BlockSpec((_B, _NPOS), lambda i: (0, 0)),
            pl.BlockSpec((_T, _D), lambda i: (i, 0)),
        ],
        out_specs=[
            pl.BlockSpec((_B * _NPOS, _T), lambda i: (0, i)),
            pl.BlockSpec((_B * _NNEG, _NPOS), lambda i: (0, 0)),
            pl.BlockSpec((_T, _D), lambda i: (i, 0)),
            pl.BlockSpec((1, _NPOS), lambda i: (0, 0)),
            pl.BlockSpec((_B, _NPOS), lambda i: (0, 0)),
        ],
        out_shape=[
            jax.ShapeDtypeStruct((_B * _NPOS, _NLEM), jnp.float32),
            jax.ShapeDtypeStruct((_B * _NNEG, _NPOS), jnp.float32),
            jax.ShapeDtypeStruct((_NLEM, _D), jnp.float32),
            jax.ShapeDtypeStruct((1, _NPOS), jnp.int32),
            jax.ShapeDtypeStruct((_B, _NPOS), jnp.int32),
        ],
    )(x, y.astype(jnp.int32), visible, memory)

    similarity = sim.reshape(_B, _NPOS, _NLEM)
    noise_similarity = noise.reshape(_B, _NNEG, _NPOS)
    accumulate_delta = acc.reshape(_NPOS)
    return (similarity, y_idx, noise_similarity, new_memory, accumulate_delta)
